# baseline (speedup 1.0000x reference)
import jax
import jax.numpy as jnp
from jax.experimental import pallas as pl
from jax.experimental.pallas import tpu as pltpu

N = 10000
D_IN = 128
D_OUT = 128
HALF = N // 2
M_BLK = 200
N_STEPS = HALF // M_BLK


def _fused_kernel(eye_ref, x_ref, w_ref, a0_ref, a1_ref, b_ref, o_ref, h_ref):
    i = pl.program_id(0)

    @pl.when(i == 0)
    def _():
        h_ref[...] = jnp.dot(x_ref[...], w_ref[...],
                             preferred_element_type=jnp.float32)

    eye_on = eye_ref[0, 0] != 0
    bias = b_ref[...]

    p0 = jnp.dot(a0_ref[...], h_ref[...], preferred_element_type=jnp.float32)
    h0 = h_ref[pl.ds(i * M_BLK, M_BLK), :]
    o_ref[0, :, :] = jnp.where(eye_on, h0, p0) + bias

    p1 = jnp.dot(a1_ref[...], h_ref[...], preferred_element_type=jnp.float32)
    h1 = h_ref[pl.ds(HALF + i * M_BLK, M_BLK), :]
    o_ref[1, :, :] = jnp.where(eye_on, h1, p1) + bias


def kernel(input, adj, shared_weight, output_bias, eye):
    eye_arr = jnp.asarray(eye, jnp.int32).reshape(1, 1)
    bias2d = output_bias.reshape(1, D_OUT)

    y2 = pl.pallas_call(
        _fused_kernel,
        grid=(N_STEPS,),
        out_shape=jax.ShapeDtypeStruct((2, HALF, D_OUT), jnp.float32),
        in_specs=[
            pl.BlockSpec((1, 1), lambda i: (0, 0)),
            pl.BlockSpec((N, D_IN), lambda i: (0, 0)),
            pl.BlockSpec((D_IN, D_OUT), lambda i: (0, 0)),
            pl.BlockSpec((M_BLK, N), lambda i: (i, 0)),
            pl.BlockSpec((M_BLK, N), lambda i: (i + N_STEPS, 0)),
            pl.BlockSpec((1, D_OUT), lambda i: (0, 0)),
        ],
        out_specs=pl.BlockSpec((2, M_BLK, D_OUT), lambda i: (0, i, 0)),
        scratch_shapes=[pltpu.VMEM((N, D_OUT), jnp.float32)],
        compiler_params=pltpu.CompilerParams(
            dimension_semantics=("arbitrary",),
        ),
    )(eye_arr, input, shared_weight, adj, adj, bias2d)

    return y2.reshape(N, D_OUT)


# speedup vs baseline: 1.0028x; 1.0028x over previous
import jax
import jax.numpy as jnp
from jax.experimental import pallas as pl
from jax.experimental.pallas import tpu as pltpu

N = 10000
D_IN = 128
D_OUT = 128
M_BLK = 400


def _fused_kernel(eye_ref, x_ref, w_ref, adj_ref, b_ref, o_ref, h_ref):
    i = pl.program_id(0)

    @pl.when(i == 0)
    def _():
        h_ref[...] = jnp.dot(x_ref[...], w_ref[...],
                             preferred_element_type=jnp.float32)

    prop = jnp.dot(adj_ref[...], h_ref[...],
                   preferred_element_type=jnp.float32)
    h_blk = h_ref[pl.ds(i * M_BLK, M_BLK), :]
    o_ref[...] = jnp.where(eye_ref[0, 0] != 0, h_blk, prop) + b_ref[...]


def kernel(input, adj, shared_weight, output_bias, eye):
    eye_arr = jnp.asarray(eye, jnp.int32).reshape(1, 1)
    bias2d = output_bias.reshape(1, D_OUT)

    return pl.pallas_call(
        _fused_kernel,
        grid=(N // M_BLK,),
        out_shape=jax.ShapeDtypeStruct((N, D_OUT), jnp.float32),
        in_specs=[
            pl.BlockSpec((1, 1), lambda i: (0, 0)),
            pl.BlockSpec((N, D_IN), lambda i: (0, 0)),
            pl.BlockSpec((D_IN, D_OUT), lambda i: (0, 0)),
            pl.BlockSpec((M_BLK, N), lambda i: (i, 0)),
            pl.BlockSpec((1, D_OUT), lambda i: (0, 0)),
        ],
        out_specs=pl.BlockSpec((M_BLK, D_OUT), lambda i: (i, 0)),
        scratch_shapes=[pltpu.VMEM((N, D_OUT), jnp.float32)],
        compiler_params=pltpu.CompilerParams(
            dimension_semantics=("arbitrary",),
        ),
    )(eye_arr, input, shared_weight, adj, bias2d)
